# Initial kernel scaffold; baseline (speedup 1.0000x reference)
import jax
import jax.numpy as jnp
from jax.experimental import pallas as pl


def kernel(x, W):
    raise NotImplementedError("write your pallas kernel here")


# speedup vs baseline: 2.8918x; 2.8918x over previous
import functools

import jax
import jax.numpy as jnp
from jax.experimental import pallas as pl
from jax.experimental.pallas import tpu as pltpu

N = 16384
D = 4096
E = 64
K = 8
BLOCK = 256


def _gate_body(x_ref, w_ref, fw_ref, idx_ref):
    logits = jnp.dot(x_ref[...], w_ref[...].T, preferred_element_type=jnp.float32)
    iota_e = jax.lax.broadcasted_iota(jnp.int32, logits.shape, 1)

    work = logits
    vals = []
    idxs = []
    for _ in range(K):
        m = jnp.max(work, axis=1, keepdims=True)
        idx = jnp.min(jnp.where(work == m, iota_e, E), axis=1, keepdims=True)
        vals.append(m)
        idxs.append(idx)
        work = jnp.where(iota_e == idx, -jnp.inf, work)

    v = jnp.concatenate(vals, axis=1)
    e = jnp.exp(v - v[:, 0:1])
    w8 = e / jnp.sum(e, axis=1, keepdims=True)

    full = jnp.zeros_like(logits)
    for k in range(K):
        full = full + jnp.where(iota_e == idxs[k], w8[:, k : k + 1], 0.0)

    fw_ref[...] = full
    idx_ref[...] = jnp.concatenate(idxs, axis=1)


@functools.partial(jax.jit, static_argnames=("interpret",))
def _gate(x, W, interpret=False):
    grid = (N // BLOCK,)
    return pl.pallas_call(
        _gate_body,
        grid=grid,
        in_specs=[
            pl.BlockSpec((BLOCK, D), lambda i: (i, 0)),
            pl.BlockSpec((E, D), lambda i: (0, 0)),
        ],
        out_specs=[
            pl.BlockSpec((BLOCK, E), lambda i: (i, 0)),
            pl.BlockSpec((BLOCK, K), lambda i: (i, 0)),
        ],
        out_shape=[
            jax.ShapeDtypeStruct((N, E), jnp.float32),
            jax.ShapeDtypeStruct((N, K), jnp.int32),
        ],
        compiler_params=pltpu.CompilerParams(
            dimension_semantics=("arbitrary",),
        ),
        interpret=interpret,
    )(x, W)


def kernel(x, W):
    full_weights, top_k_indices = _gate(x, W)
    return (full_weights, top_k_indices)


# speedup vs baseline: 5.0701x; 1.7533x over previous
import functools

import jax
import jax.numpy as jnp
from jax.experimental import pallas as pl
from jax.experimental.pallas import tpu as pltpu

N = 16384
D = 4096
E = 64
K = 8
BLOCK = 256


def _gate_body(x_ref, w_ref, fw_ref, idx_ref):
    lt = jax.lax.dot_general(
        w_ref[...], x_ref[...],
        (((1,), (1,)), ((), ())),
        preferred_element_type=jnp.float32,
    )
    iota_e = jax.lax.broadcasted_iota(jnp.int32, lt.shape, 0)

    work = lt
    vals = []
    idxs = []
    for _ in range(K):
        m = jnp.max(work, axis=0, keepdims=True)
        idx = jnp.min(jnp.where(work == m, iota_e, E), axis=0, keepdims=True)
        vals.append(m)
        idxs.append(idx)
        work = jnp.where(iota_e == idx, -jnp.inf, work)

    v = jnp.concatenate(vals, axis=0)
    e = jnp.exp(v - v[0:1, :])
    w8 = e / jnp.sum(e, axis=0, keepdims=True)

    full = jnp.zeros_like(lt)
    for k in range(K):
        full = full + jnp.where(iota_e == idxs[k], w8[k : k + 1, :], 0.0)

    fw_ref[...] = full.T
    idx_ref[...] = jnp.concatenate(idxs, axis=0).T


@functools.partial(jax.jit, static_argnames=("interpret",))
def _gate(x, W, interpret=False):
    grid = (N // BLOCK,)
    return pl.pallas_call(
        _gate_body,
        grid=grid,
        in_specs=[
            pl.BlockSpec((BLOCK, D), lambda i: (i, 0)),
            pl.BlockSpec((E, D), lambda i: (0, 0)),
        ],
        out_specs=[
            pl.BlockSpec((BLOCK, E), lambda i: (i, 0)),
            pl.BlockSpec((BLOCK, K), lambda i: (i, 0)),
        ],
        out_shape=[
            jax.ShapeDtypeStruct((N, E), jnp.float32),
            jax.ShapeDtypeStruct((N, K), jnp.int32),
        ],
        compiler_params=pltpu.CompilerParams(
            dimension_semantics=("arbitrary",),
        ),
        interpret=interpret,
    )(x, W)


def kernel(x, W):
    full_weights, top_k_indices = _gate(x, W)
    return (full_weights, top_k_indices)


# speedup vs baseline: 6.0625x; 1.1957x over previous
import functools

import jax
import jax.numpy as jnp
from jax.experimental import pallas as pl
from jax.experimental.pallas import tpu as pltpu

N = 16384
D = 4096
E = 64
K = 8
BLOCK = 512


def _gate_body(x_ref, w_ref, fw_ref, idx_ref):
    lt = jax.lax.dot_general(
        w_ref[...], x_ref[...],
        (((1,), (1,)), ((), ())),
        preferred_element_type=jnp.float32,
    )
    iota_e = jax.lax.broadcasted_iota(jnp.int32, lt.shape, 0)

    work = lt
    vals = []
    idxs = []
    for _ in range(K):
        m = jnp.max(work, axis=0, keepdims=True)
        idx = jnp.min(jnp.where(work == m, iota_e, E), axis=0, keepdims=True)
        vals.append(m)
        idxs.append(idx)
        work = jnp.where(iota_e == idx, -jnp.inf, work)

    v = jnp.concatenate(vals, axis=0)
    e = jnp.exp(v - v[0:1, :])
    w8 = e / jnp.sum(e, axis=0, keepdims=True)

    full = jnp.zeros_like(lt)
    for k in range(K):
        full = full + jnp.where(iota_e == idxs[k], w8[k : k + 1, :], 0.0)

    fw_ref[...] = full.T
    idx_ref[...] = jnp.concatenate(idxs, axis=0).T


@functools.partial(jax.jit, static_argnames=("interpret",))
def _gate(x, W, interpret=False):
    grid = (N // BLOCK,)
    return pl.pallas_call(
        _gate_body,
        grid=grid,
        in_specs=[
            pl.BlockSpec((BLOCK, D), lambda i: (i, 0)),
            pl.BlockSpec((E, D), lambda i: (0, 0)),
        ],
        out_specs=[
            pl.BlockSpec((BLOCK, E), lambda i: (i, 0)),
            pl.BlockSpec((BLOCK, K), lambda i: (i, 0)),
        ],
        out_shape=[
            jax.ShapeDtypeStruct((N, E), jnp.float32),
            jax.ShapeDtypeStruct((N, K), jnp.int32),
        ],
        compiler_params=pltpu.CompilerParams(
            dimension_semantics=("arbitrary",),
        ),
        interpret=interpret,
    )(x, W)


def kernel(x, W):
    full_weights, top_k_indices = _gate(x, W)
    return (full_weights, top_k_indices)


# speedup vs baseline: 6.4982x; 1.0719x over previous
import functools

import jax
import jax.numpy as jnp
from jax.experimental import pallas as pl
from jax.experimental.pallas import tpu as pltpu

N = 16384
D = 4096
E = 64
K = 8
BLOCK = 1024


def _gate_body(x_ref, w_ref, fw_ref, idx_ref):
    lt = jax.lax.dot_general(
        w_ref[...], x_ref[...],
        (((1,), (1,)), ((), ())),
        preferred_element_type=jnp.float32,
    )
    iota_e = jax.lax.broadcasted_iota(jnp.int32, lt.shape, 0)

    work = lt
    vals = []
    idxs = []
    for _ in range(K):
        m = jnp.max(work, axis=0, keepdims=True)
        idx = jnp.min(jnp.where(work == m, iota_e, E), axis=0, keepdims=True)
        vals.append(m)
        idxs.append(idx)
        work = jnp.where(iota_e == idx, -jnp.inf, work)

    v = jnp.concatenate(vals, axis=0)
    e = jnp.exp(v - v[0:1, :])
    w8 = e / jnp.sum(e, axis=0, keepdims=True)

    full = jnp.zeros_like(lt)
    for k in range(K):
        full = full + jnp.where(iota_e == idxs[k], w8[k : k + 1, :], 0.0)

    fw_ref[...] = full.T
    idx_ref[...] = jnp.concatenate(idxs, axis=0).T


@functools.partial(jax.jit, static_argnames=("interpret",))
def _gate(x, W, interpret=False):
    grid = (N // BLOCK,)
    return pl.pallas_call(
        _gate_body,
        grid=grid,
        in_specs=[
            pl.BlockSpec((BLOCK, D), lambda i: (i, 0)),
            pl.BlockSpec((E, D), lambda i: (0, 0)),
        ],
        out_specs=[
            pl.BlockSpec((BLOCK, E), lambda i: (i, 0)),
            pl.BlockSpec((BLOCK, K), lambda i: (i, 0)),
        ],
        out_shape=[
            jax.ShapeDtypeStruct((N, E), jnp.float32),
            jax.ShapeDtypeStruct((N, K), jnp.int32),
        ],
        compiler_params=pltpu.CompilerParams(
            dimension_semantics=("arbitrary",),
        ),
        interpret=interpret,
    )(x, W)


def kernel(x, W):
    full_weights, top_k_indices = _gate(x, W)
    return (full_weights, top_k_indices)


# speedup vs baseline: 6.5115x; 1.0020x over previous
import functools

import jax
import jax.numpy as jnp
from jax.experimental import pallas as pl
from jax.experimental.pallas import tpu as pltpu

N = 16384
D = 4096
E = 64
K = 8
BLOCK = 1024
NSPLIT = 4
DC = D // NSPLIT


def _argmax_tree(val, idx):
    while val.shape[0] > 1:
        h = val.shape[0] // 2
        a, b = val[:h], val[h:]
        ia, ib = idx[:h], idx[h:]
        take_a = a >= b
        val = jnp.where(take_a, a, b)
        idx = jnp.where(take_a, ia, ib)
    return val, idx


def _gate_body(*refs):
    x_refs = refs[:NSPLIT]
    w_ref = refs[NSPLIT]
    fw_ref, idx_ref = refs[NSPLIT + 1], refs[NSPLIT + 2]

    lt = jnp.zeros((E, BLOCK), dtype=jnp.float32)
    for c in range(NSPLIT):
        lt = lt + jax.lax.dot_general(
            w_ref[:, c * DC : (c + 1) * DC], x_refs[c][...],
            (((1,), (1,)), ((), ())),
            preferred_element_type=jnp.float32,
        )
    iota_e = jax.lax.broadcasted_iota(jnp.int32, lt.shape, 0)

    work = lt
    vals = []
    idxs = []
    for _ in range(K):
        m, idx = _argmax_tree(work, iota_e)
        vals.append(m)
        idxs.append(idx)
        work = jnp.where(iota_e == idx, -jnp.inf, work)

    v = jnp.concatenate(vals, axis=0)
    e = jnp.exp(v - v[0:1, :])
    denom = jnp.sum(e, axis=0, keepdims=True)

    full = jnp.where(
        lt >= v[K - 1 : K, :],
        jnp.exp(lt - v[0:1, :]) / denom,
        0.0,
    )

    fw_ref[...] = full.T
    idx_ref[...] = jnp.concatenate(idxs, axis=0).T


@functools.partial(jax.jit, static_argnames=("interpret",))
def _gate(x, W, interpret=False):
    grid = (N // BLOCK,)
    return pl.pallas_call(
        _gate_body,
        grid=grid,
        in_specs=[
            pl.BlockSpec((BLOCK, DC), lambda i, c=c: (i, c)) for c in range(NSPLIT)
        ]
        + [pl.BlockSpec((E, D), lambda i: (0, 0))],
        out_specs=[
            pl.BlockSpec((BLOCK, E), lambda i: (i, 0)),
            pl.BlockSpec((BLOCK, K), lambda i: (i, 0)),
        ],
        out_shape=[
            jax.ShapeDtypeStruct((N, E), jnp.float32),
            jax.ShapeDtypeStruct((N, K), jnp.int32),
        ],
        compiler_params=pltpu.CompilerParams(
            dimension_semantics=("arbitrary",),
        ),
        interpret=interpret,
    )(*([x] * NSPLIT), W)


def kernel(x, W):
    full_weights, top_k_indices = _gate(x, W)
    return (full_weights, top_k_indices)
